# baseline (speedup 1.0000x reference)
import functools

import jax
import jax.numpy as jnp
from jax import lax
from jax.experimental import pallas as pl
from jax.experimental.pallas import tpu as pltpu
from jax.experimental.pallas import tpu_sc as plsc

VOCAB = 100000
EMBED = 256
HID = 512
H = HID // 2
G4 = 4 * H
ATTN = 256
TAG = 32
B = 128
L = 200

SC_CORES = 2
SC_SUBCORES = 16
NW = SC_CORES * SC_SUBCORES
ROWS = B * L
ROWS_PER_W = ROWS // NW
GCH = 80
NGCH = ROWS_PER_W // GCH

C = 8
NC = L // C

C2 = 40
NC2 = L // C2


def _sc_gather(emb, idx):
    mesh = plsc.VectorSubcoreMesh(core_axis_name="c", subcore_axis_name="s")

    @functools.partial(
        pl.kernel,
        mesh=mesh,
        out_type=jax.ShapeDtypeStruct((ROWS, EMBED), jnp.float32),
        scratch_types=[
            pltpu.VMEM((GCH,), jnp.int32),
            pltpu.VMEM((GCH, EMBED), jnp.float32),
            pltpu.SemaphoreType.DMA,
        ],
    )
    def gather_k(emb_hbm, idx_hbm, out_hbm, idx_v, rows_v, sem):
        wid = lax.axis_index("s") * SC_CORES + lax.axis_index("c")
        base = wid * ROWS_PER_W

        def body(c, carry):
            off = base + c * GCH
            pltpu.sync_copy(idx_hbm.at[pl.ds(off, GCH)], idx_v)
            pltpu.async_copy(emb_hbm.at[idx_v], rows_v, sem).wait()
            pltpu.sync_copy(rows_v, out_hbm.at[pl.ds(off, GCH)])
            return carry

        lax.fori_loop(0, NGCH, body, 0)

    return gather_k(emb, idx)


def _lstm_body(xef_ref, xeb_ref, wif_ref, whf_ref, bf_ref, wib_ref, whb_ref,
               bb_ref, hf_out, hb_out, hfs, cfs, hbs, cbs):
    g = pl.program_id(0)

    @pl.when(g == 0)
    def _():
        z = jnp.zeros((B, H), jnp.float32)
        hfs[...] = z
        cfs[...] = z
        hbs[...] = z
        cbs[...] = z

    whf = whf_ref[...]
    bf = bf_ref[...]
    whb = whb_ref[...]
    bb = bb_ref[...]

    hf = hfs[...]
    cf = cfs[...]
    hb = hbs[...]
    cb = cbs[...]

    xef = xef_ref[...].astype(jnp.bfloat16).reshape(B * C, EMBED)
    xeb = xeb_ref[...].astype(jnp.bfloat16).reshape(B * C, EMBED)
    ginf = jnp.dot(xef, wif_ref[...],
                   preferred_element_type=jnp.float32).reshape(B, C, G4)
    ginb = jnp.dot(xeb, wib_ref[...],
                   preferred_element_type=jnp.float32).reshape(B, C, G4)

    def cell(gin, h, c, wh, b):
        gates = gin + b + jnp.dot(h.astype(jnp.bfloat16), wh,
                                  preferred_element_type=jnp.float32)
        i = jax.nn.sigmoid(gates[:, 0 * H:1 * H])
        f = jax.nn.sigmoid(gates[:, 1 * H:2 * H])
        gg = jnp.tanh(gates[:, 2 * H:3 * H])
        o = jax.nn.sigmoid(gates[:, 3 * H:4 * H])
        c_new = f * c + i * gg
        h_new = o * jnp.tanh(c_new)
        return h_new, c_new

    for t in range(C):
        tb = C - 1 - t
        hf, cf = cell(ginf[:, t, :], hf, cf, whf, bf)
        hb, cb = cell(ginb[:, tb, :], hb, cb, whb, bb)
        hf_out[:, t, :] = hf
        hb_out[:, tb, :] = hb

    hfs[...] = hf
    cfs[...] = cf
    hbs[...] = hb
    cbs[...] = cb


def _lstm(xe, wif, whf, bf, wib, whb, bb):
    return pl.pallas_call(
        _lstm_body,
        grid=(NC,),
        in_specs=[
            pl.BlockSpec((B, C, EMBED), lambda g: (0, g, 0)),
            pl.BlockSpec((B, C, EMBED), lambda g: (0, NC - 1 - g, 0)),
            pl.BlockSpec((EMBED, G4), lambda g: (0, 0)),
            pl.BlockSpec((H, G4), lambda g: (0, 0)),
            pl.BlockSpec((1, G4), lambda g: (0, 0)),
            pl.BlockSpec((EMBED, G4), lambda g: (0, 0)),
            pl.BlockSpec((H, G4), lambda g: (0, 0)),
            pl.BlockSpec((1, G4), lambda g: (0, 0)),
        ],
        out_specs=[
            pl.BlockSpec((B, C, H), lambda g: (0, g, 0)),
            pl.BlockSpec((B, C, H), lambda g: (0, NC - 1 - g, 0)),
        ],
        out_shape=[
            jax.ShapeDtypeStruct((B, L, H), jnp.float32),
            jax.ShapeDtypeStruct((B, L, H), jnp.float32),
        ],
        scratch_shapes=[pltpu.VMEM((B, H), jnp.float32)] * 4,
        compiler_params=pltpu.CompilerParams(
            dimension_semantics=("arbitrary",)),
    )(xe, xe, wif, whf, bf, wib, whb, bb)


def _attn_body(hf_ref, hb_ref, waf_ref, wab_ref, v_ref, wof_ref, wob_ref,
               bo_ref, out_ref, sc_ref, accf_ref, accb_ref):
    p = pl.program_id(0)
    g = pl.program_id(1)

    @pl.when((p == 0) & (g == 0))
    def _():
        accf_ref[...] = jnp.zeros((B, H), jnp.float32)
        accb_ref[...] = jnp.zeros((B, H), jnp.float32)

    @pl.when(p == 0)
    def _():
        waf = waf_ref[...]
        wab = wab_ref[...]
        v = v_ref[...]
        cols = []
        for t in range(C2):
            u = jnp.tanh(
                jnp.dot(hf_ref[:, t, :].astype(jnp.bfloat16), waf,
                        preferred_element_type=jnp.float32)
                + jnp.dot(hb_ref[:, t, :].astype(jnp.bfloat16), wab,
                          preferred_element_type=jnp.float32))
            cols.append(jnp.sum(u * v, axis=1, keepdims=True))
        sc_ref[g] = jnp.concatenate(cols, axis=1)

        @pl.when(g == NC2 - 1)
        def _():
            s_all = sc_ref[...]
            m = jnp.max(jnp.max(s_all, axis=0), axis=1, keepdims=True)
            w = jnp.exp(s_all - m[None, :, :])
            z = jnp.sum(jnp.sum(w, axis=0), axis=1, keepdims=True)
            sc_ref[...] = w / z[None, :, :]

    @pl.when(p == 1)
    def _():
        accf = accf_ref[...]
        accb = accb_ref[...]
        wg = sc_ref[g]
        for t in range(C2):
            w_col = wg[:, t:t + 1]
            accf = accf + w_col * hf_ref[:, t, :]
            accb = accb + w_col * hb_ref[:, t, :]
        accf_ref[...] = accf
        accb_ref[...] = accb

        @pl.when(g == NC2 - 1)
        def _():
            out_ref[...] = (
                jnp.dot(accf, wof_ref[...],
                        preferred_element_type=jnp.float32)
                + jnp.dot(accb, wob_ref[...],
                          preferred_element_type=jnp.float32)
                + bo_ref[...])


def _attn(hf, hb, waf, wab, v, wof, wob, bo):
    return pl.pallas_call(
        _attn_body,
        grid=(2, NC2),
        in_specs=[
            pl.BlockSpec((B, C2, H), lambda p, g: (0, g, 0)),
            pl.BlockSpec((B, C2, H), lambda p, g: (0, g, 0)),
            pl.BlockSpec((H, ATTN), lambda p, g: (0, 0)),
            pl.BlockSpec((H, ATTN), lambda p, g: (0, 0)),
            pl.BlockSpec((1, ATTN), lambda p, g: (0, 0)),
            pl.BlockSpec((H, TAG), lambda p, g: (0, 0)),
            pl.BlockSpec((H, TAG), lambda p, g: (0, 0)),
            pl.BlockSpec((1, TAG), lambda p, g: (0, 0)),
        ],
        out_specs=pl.BlockSpec((B, TAG), lambda p, g: (0, 0)),
        out_shape=jax.ShapeDtypeStruct((B, TAG), jnp.float32),
        scratch_shapes=[
            pltpu.VMEM((NC2, B, C2), jnp.float32),
            pltpu.VMEM((B, H), jnp.float32),
            pltpu.VMEM((B, H), jnp.float32),
        ],
        compiler_params=pltpu.CompilerParams(
            dimension_semantics=("arbitrary", "arbitrary")),
    )(hf, hb, waf, wab, v, wof, wob, bo)


def kernel(x, emb, Wih_f, Whh_f, bih_f, bhh_f, Wih_b, Whh_b, bih_b, bhh_b,
           W_attn, v_attn, W_out, b_out):
    idx = x.reshape(-1).astype(jnp.int32)
    xe = _sc_gather(emb, idx).reshape(B, L, EMBED)

    bf16 = jnp.bfloat16
    hf, hb = _lstm(
        xe,
        Wih_f.T.astype(bf16), Whh_f.T.astype(bf16),
        (bih_f + bhh_f)[None, :],
        Wih_b.T.astype(bf16), Whh_b.T.astype(bf16),
        (bih_b + bhh_b)[None, :],
    )

    return _attn(
        hf, hb,
        W_attn[:H, :].astype(bf16), W_attn[H:, :].astype(bf16),
        v_attn[None, :],
        W_out[:, :H].T, W_out[:, H:].T, b_out[None, :],
    )


# speedup vs baseline: 2.6034x; 2.6034x over previous
import functools

import jax
import jax.numpy as jnp
from jax import lax
from jax.experimental import pallas as pl
from jax.experimental.pallas import tpu as pltpu
from jax.experimental.pallas import tpu_sc as plsc

VOCAB = 100000
EMBED = 256
HID = 512
H = HID // 2
G4 = 4 * H
ATTN = 256
TAG = 32
B = 128
L = 200

SC_CORES = 2
SC_SUBCORES = 16
NW = SC_CORES * SC_SUBCORES
ROWS = B * L
ROWS_PER_W = ROWS // NW
GCH = 80
NGCH = ROWS_PER_W // GCH

C = 8
NC = L // C

C2 = 40
NC2 = L // C2


def _sc_gather(emb, idx):
    mesh = plsc.VectorSubcoreMesh(core_axis_name="c", subcore_axis_name="s")

    @functools.partial(
        pl.kernel,
        mesh=mesh,
        out_type=jax.ShapeDtypeStruct((ROWS, EMBED), jnp.float32),
        scratch_types=[
            pltpu.VMEM((GCH,), jnp.int32),
            pltpu.VMEM((GCH, EMBED), jnp.float32),
            pltpu.SemaphoreType.DMA,
        ],
    )
    def gather_k(emb_hbm, idx_hbm, out_hbm, idx_v, rows_v, sem):
        wid = lax.axis_index("s") * SC_CORES + lax.axis_index("c")
        base = wid * ROWS_PER_W

        def body(c, carry):
            off = base + c * GCH
            pltpu.sync_copy(idx_hbm.at[pl.ds(off, GCH)], idx_v)
            pltpu.async_copy(emb_hbm.at[idx_v], rows_v, sem).wait()
            pltpu.sync_copy(rows_v, out_hbm.at[pl.ds(off, GCH)])
            return carry

        lax.fori_loop(0, NGCH, body, 0)

    return gather_k(emb, idx)


def _lstm_body(xef_ref, xeb_ref, wif_ref, whf_ref, bf_ref, wib_ref, whb_ref,
               bb_ref, hf_out, hb_out, hfs, cfs, hbs, cbs):
    g = pl.program_id(0)

    @pl.when(g == 0)
    def _():
        z = jnp.zeros((B, H), jnp.float32)
        hfs[...] = z
        cfs[...] = z
        hbs[...] = z
        cbs[...] = z

    whf = whf_ref[...]
    bf = bf_ref[...]
    whb = whb_ref[...]
    bb = bb_ref[...]

    hf = hfs[...]
    cf = cfs[...]
    hb = hbs[...]
    cb = cbs[...]

    wif = wif_ref[...]
    wib = wib_ref[...]

    def cell(x, h, c, wi, wh, b):
        gin = jnp.dot(x.astype(jnp.bfloat16), wi,
                      preferred_element_type=jnp.float32)
        gates = gin + b + jnp.dot(h.astype(jnp.bfloat16), wh,
                                  preferred_element_type=jnp.float32)
        i = jax.nn.sigmoid(gates[:, 0 * H:1 * H])
        f = jax.nn.sigmoid(gates[:, 1 * H:2 * H])
        gg = jnp.tanh(gates[:, 2 * H:3 * H])
        o = jax.nn.sigmoid(gates[:, 3 * H:4 * H])
        c_new = f * c + i * gg
        h_new = o * jnp.tanh(c_new)
        return h_new, c_new

    for t in range(C):
        tb = C - 1 - t
        hf, cf = cell(xef_ref[:, t, :], hf, cf, wif, whf, bf)
        hb, cb = cell(xeb_ref[:, tb, :], hb, cb, wib, whb, bb)
        hf_out[:, t, :] = hf
        hb_out[:, tb, :] = hb

    hfs[...] = hf
    cfs[...] = cf
    hbs[...] = hb
    cbs[...] = cb


def _lstm(xe, wif, whf, bf, wib, whb, bb):
    return pl.pallas_call(
        _lstm_body,
        grid=(NC,),
        in_specs=[
            pl.BlockSpec((B, C, EMBED), lambda g: (0, g, 0)),
            pl.BlockSpec((B, C, EMBED), lambda g: (0, NC - 1 - g, 0)),
            pl.BlockSpec((EMBED, G4), lambda g: (0, 0)),
            pl.BlockSpec((H, G4), lambda g: (0, 0)),
            pl.BlockSpec((1, G4), lambda g: (0, 0)),
            pl.BlockSpec((EMBED, G4), lambda g: (0, 0)),
            pl.BlockSpec((H, G4), lambda g: (0, 0)),
            pl.BlockSpec((1, G4), lambda g: (0, 0)),
        ],
        out_specs=[
            pl.BlockSpec((B, C, H), lambda g: (0, g, 0)),
            pl.BlockSpec((B, C, H), lambda g: (0, NC - 1 - g, 0)),
        ],
        out_shape=[
            jax.ShapeDtypeStruct((B, L, H), jnp.float32),
            jax.ShapeDtypeStruct((B, L, H), jnp.float32),
        ],
        scratch_shapes=[pltpu.VMEM((B, H), jnp.float32)] * 4,
        compiler_params=pltpu.CompilerParams(
            dimension_semantics=("arbitrary",)),
    )(xe, xe, wif, whf, bf, wib, whb, bb)


def _attn_body(hf_ref, hb_ref, waf_ref, wab_ref, v_ref, wof_ref, wob_ref,
               bo_ref, out_ref, sc_ref, accf_ref, accb_ref):
    p = pl.program_id(0)
    g = pl.program_id(1)

    @pl.when((p == 0) & (g == 0))
    def _():
        accf_ref[...] = jnp.zeros((B, H), jnp.float32)
        accb_ref[...] = jnp.zeros((B, H), jnp.float32)

    @pl.when(p == 0)
    def _():
        waf = waf_ref[...]
        wab = wab_ref[...]
        v = v_ref[...]
        cols = []
        for t in range(C2):
            u = jnp.tanh(
                jnp.dot(hf_ref[:, t, :].astype(jnp.bfloat16), waf,
                        preferred_element_type=jnp.float32)
                + jnp.dot(hb_ref[:, t, :].astype(jnp.bfloat16), wab,
                          preferred_element_type=jnp.float32))
            cols.append(jnp.sum(u * v, axis=1, keepdims=True))
        sc_ref[g] = jnp.concatenate(cols, axis=1)

        @pl.when(g == NC2 - 1)
        def _():
            s_all = sc_ref[...]
            m = jnp.max(jnp.max(s_all, axis=0), axis=1, keepdims=True)
            w = jnp.exp(s_all - m[None, :, :])
            z = jnp.sum(jnp.sum(w, axis=0), axis=1, keepdims=True)
            sc_ref[...] = w / z[None, :, :]

    @pl.when(p == 1)
    def _():
        accf = accf_ref[...]
        accb = accb_ref[...]
        wg = sc_ref[g]
        for t in range(C2):
            w_col = wg[:, t:t + 1]
            accf = accf + w_col * hf_ref[:, t, :]
            accb = accb + w_col * hb_ref[:, t, :]
        accf_ref[...] = accf
        accb_ref[...] = accb

        @pl.when(g == NC2 - 1)
        def _():
            out_ref[...] = (
                jnp.dot(accf, wof_ref[...],
                        preferred_element_type=jnp.float32)
                + jnp.dot(accb, wob_ref[...],
                          preferred_element_type=jnp.float32)
                + bo_ref[...])


def _attn(hf, hb, waf, wab, v, wof, wob, bo):
    return pl.pallas_call(
        _attn_body,
        grid=(2, NC2),
        in_specs=[
            pl.BlockSpec((B, C2, H), lambda p, g: (0, g, 0)),
            pl.BlockSpec((B, C2, H), lambda p, g: (0, g, 0)),
            pl.BlockSpec((H, ATTN), lambda p, g: (0, 0)),
            pl.BlockSpec((H, ATTN), lambda p, g: (0, 0)),
            pl.BlockSpec((1, ATTN), lambda p, g: (0, 0)),
            pl.BlockSpec((H, TAG), lambda p, g: (0, 0)),
            pl.BlockSpec((H, TAG), lambda p, g: (0, 0)),
            pl.BlockSpec((1, TAG), lambda p, g: (0, 0)),
        ],
        out_specs=pl.BlockSpec((B, TAG), lambda p, g: (0, 0)),
        out_shape=jax.ShapeDtypeStruct((B, TAG), jnp.float32),
        scratch_shapes=[
            pltpu.VMEM((NC2, B, C2), jnp.float32),
            pltpu.VMEM((B, H), jnp.float32),
            pltpu.VMEM((B, H), jnp.float32),
        ],
        compiler_params=pltpu.CompilerParams(
            dimension_semantics=("arbitrary", "arbitrary")),
    )(hf, hb, waf, wab, v, wof, wob, bo)


def kernel(x, emb, Wih_f, Whh_f, bih_f, bhh_f, Wih_b, Whh_b, bih_b, bhh_b,
           W_attn, v_attn, W_out, b_out):
    idx = x.reshape(-1).astype(jnp.int32)
    xe = _sc_gather(emb, idx).reshape(B, L, EMBED)

    bf16 = jnp.bfloat16
    hf, hb = _lstm(
        xe,
        Wih_f.T.astype(bf16), Whh_f.T.astype(bf16),
        (bih_f + bhh_f)[None, :],
        Wih_b.T.astype(bf16), Whh_b.T.astype(bf16),
        (bih_b + bhh_b)[None, :],
    )

    return _attn(
        hf, hb,
        W_attn[:H, :].astype(bf16), W_attn[H:, :].astype(bf16),
        v_attn[None, :],
        W_out[:, :H].T, W_out[:, H:].T, b_out[None, :],
    )


# speedup vs baseline: 3.5980x; 1.3820x over previous
import functools

import jax
import jax.numpy as jnp
from jax import lax
from jax.experimental import pallas as pl
from jax.experimental.pallas import tpu as pltpu
from jax.experimental.pallas import tpu_sc as plsc

VOCAB = 100000
EMBED = 256
HID = 512
H = HID // 2
G4 = 4 * H
ATTN = 256
TAG = 32
B = 128
L = 200

SC_CORES = 2
SC_SUBCORES = 16
NW = SC_CORES * SC_SUBCORES
ROWS = B * L
ROWS_PER_W = ROWS // NW
GCH = 80
NGCH = ROWS_PER_W // GCH

C = 8
NC = L // C

C2 = 40
NC2 = L // C2


def _sc_gather(emb, idx):
    mesh = plsc.VectorSubcoreMesh(core_axis_name="c", subcore_axis_name="s")

    @functools.partial(
        pl.kernel,
        mesh=mesh,
        out_type=jax.ShapeDtypeStruct((ROWS, EMBED), jnp.float32),
        scratch_types=[
            pltpu.VMEM((GCH,), jnp.int32),
            pltpu.VMEM((GCH, EMBED), jnp.float32),
            pltpu.SemaphoreType.DMA,
        ],
    )
    def gather_k(emb_hbm, idx_hbm, out_hbm, idx_v, rows_v, sem):
        wid = lax.axis_index("s") * SC_CORES + lax.axis_index("c")
        base = wid * ROWS_PER_W

        def body(c, carry):
            off = base + c * GCH
            pltpu.sync_copy(idx_hbm.at[pl.ds(off, GCH)], idx_v)
            pltpu.async_copy(emb_hbm.at[idx_v], rows_v, sem).wait()
            pltpu.sync_copy(rows_v, out_hbm.at[pl.ds(off, GCH)])
            return carry

        lax.fori_loop(0, NGCH, body, 0)

    return gather_k(emb, idx)


def _lstm_body(xef_ref, xeb_ref, wif_ref, whf_ref, bf_ref, wib_ref, whb_ref,
               bb_ref, hf_out, hb_out, hfs, cfs, hbs, cbs):
    g = pl.program_id(0)

    @pl.when(g == 0)
    def _():
        z = jnp.zeros((B, H), jnp.float32)
        hfs[...] = z
        cfs[...] = z
        hbs[...] = z
        cbs[...] = z

    whf = whf_ref[...]
    bf = bf_ref[...]
    whb = whb_ref[...]
    bb = bb_ref[...]

    hf = hfs[...]
    cf = cfs[...]
    hb = hbs[...]
    cb = cbs[...]

    wif = wif_ref[...]
    wib = wib_ref[...]

    def cell(x, h, c, wi, wh, b):
        gates = (jnp.dot(x, wi, preferred_element_type=jnp.float32) + b
                 + jnp.dot(h, wh, preferred_element_type=jnp.float32))
        i = jax.nn.sigmoid(gates[:, 0 * H:1 * H])
        f = jax.nn.sigmoid(gates[:, 1 * H:2 * H])
        gg = jnp.tanh(gates[:, 2 * H:3 * H])
        o = jax.nn.sigmoid(gates[:, 3 * H:4 * H])
        c_new = f * c + i * gg
        h_new = o * jnp.tanh(c_new)
        return h_new, c_new

    for t in range(C):
        tb = C - 1 - t
        hf, cf = cell(xef_ref[:, t, :], hf, cf, wif, whf, bf)
        hb, cb = cell(xeb_ref[:, tb, :], hb, cb, wib, whb, bb)
        hf_out[:, t, :] = hf
        hb_out[:, tb, :] = hb

    hfs[...] = hf
    cfs[...] = cf
    hbs[...] = hb
    cbs[...] = cb


def _lstm(xe, wif, whf, bf, wib, whb, bb):
    return pl.pallas_call(
        _lstm_body,
        grid=(NC,),
        in_specs=[
            pl.BlockSpec((B, C, EMBED), lambda g: (0, g, 0)),
            pl.BlockSpec((B, C, EMBED), lambda g: (0, NC - 1 - g, 0)),
            pl.BlockSpec((EMBED, G4), lambda g: (0, 0)),
            pl.BlockSpec((H, G4), lambda g: (0, 0)),
            pl.BlockSpec((1, G4), lambda g: (0, 0)),
            pl.BlockSpec((EMBED, G4), lambda g: (0, 0)),
            pl.BlockSpec((H, G4), lambda g: (0, 0)),
            pl.BlockSpec((1, G4), lambda g: (0, 0)),
        ],
        out_specs=[
            pl.BlockSpec((B, C, H), lambda g: (0, g, 0)),
            pl.BlockSpec((B, C, H), lambda g: (0, NC - 1 - g, 0)),
        ],
        out_shape=[
            jax.ShapeDtypeStruct((B, L, H), jnp.float32),
            jax.ShapeDtypeStruct((B, L, H), jnp.float32),
        ],
        scratch_shapes=[pltpu.VMEM((B, H), jnp.float32)] * 4,
        compiler_params=pltpu.CompilerParams(
            dimension_semantics=("arbitrary",)),
    )(xe, xe, wif, whf, bf, wib, whb, bb)


def _attn_body(hf_ref, hb_ref, waf_ref, wab_ref, v_ref, wof_ref, wob_ref,
               bo_ref, out_ref, sc_ref, accf_ref, accb_ref):
    p = pl.program_id(0)
    g = pl.program_id(1)

    @pl.when((p == 0) & (g == 0))
    def _():
        accf_ref[...] = jnp.zeros((B, H), jnp.float32)
        accb_ref[...] = jnp.zeros((B, H), jnp.float32)

    @pl.when(p == 0)
    def _():
        waf = waf_ref[...]
        wab = wab_ref[...]
        v = v_ref[...]
        cols = []
        for t in range(C2):
            u = jnp.tanh(
                jnp.dot(hf_ref[:, t, :], waf,
                        preferred_element_type=jnp.float32)
                + jnp.dot(hb_ref[:, t, :], wab,
                          preferred_element_type=jnp.float32))
            cols.append(jnp.sum(u * v, axis=1, keepdims=True))
        sc_ref[g] = jnp.concatenate(cols, axis=1)

        @pl.when(g == NC2 - 1)
        def _():
            s_all = sc_ref[...]
            m = jnp.max(jnp.max(s_all, axis=0), axis=1, keepdims=True)
            w = jnp.exp(s_all - m[None, :, :])
            z = jnp.sum(jnp.sum(w, axis=0), axis=1, keepdims=True)
            sc_ref[...] = w / z[None, :, :]

    @pl.when(p == 1)
    def _():
        accf = accf_ref[...]
        accb = accb_ref[...]
        wg = sc_ref[g]
        for t in range(C2):
            w_col = wg[:, t:t + 1]
            accf = accf + w_col * hf_ref[:, t, :]
            accb = accb + w_col * hb_ref[:, t, :]
        accf_ref[...] = accf
        accb_ref[...] = accb

        @pl.when(g == NC2 - 1)
        def _():
            out_ref[...] = (
                jnp.dot(accf, wof_ref[...],
                        preferred_element_type=jnp.float32)
                + jnp.dot(accb, wob_ref[...],
                          preferred_element_type=jnp.float32)
                + bo_ref[...])


def _attn(hf, hb, waf, wab, v, wof, wob, bo):
    return pl.pallas_call(
        _attn_body,
        grid=(2, NC2),
        in_specs=[
            pl.BlockSpec((B, C2, H), lambda p, g: (0, g, 0)),
            pl.BlockSpec((B, C2, H), lambda p, g: (0, g, 0)),
            pl.BlockSpec((H, ATTN), lambda p, g: (0, 0)),
            pl.BlockSpec((H, ATTN), lambda p, g: (0, 0)),
            pl.BlockSpec((1, ATTN), lambda p, g: (0, 0)),
            pl.BlockSpec((H, TAG), lambda p, g: (0, 0)),
            pl.BlockSpec((H, TAG), lambda p, g: (0, 0)),
            pl.BlockSpec((1, TAG), lambda p, g: (0, 0)),
        ],
        out_specs=pl.BlockSpec((B, TAG), lambda p, g: (0, 0)),
        out_shape=jax.ShapeDtypeStruct((B, TAG), jnp.float32),
        scratch_shapes=[
            pltpu.VMEM((NC2, B, C2), jnp.float32),
            pltpu.VMEM((B, H), jnp.float32),
            pltpu.VMEM((B, H), jnp.float32),
        ],
        compiler_params=pltpu.CompilerParams(
            dimension_semantics=("arbitrary", "arbitrary")),
    )(hf, hb, waf, wab, v, wof, wob, bo)


def kernel(x, emb, Wih_f, Whh_f, bih_f, bhh_f, Wih_b, Whh_b, bih_b, bhh_b,
           W_attn, v_attn, W_out, b_out):
    idx = x.reshape(-1).astype(jnp.int32)
    xe = _sc_gather(emb, idx).reshape(B, L, EMBED)

    hf, hb = _lstm(
        xe,
        Wih_f.T, Whh_f.T, (bih_f + bhh_f)[None, :],
        Wih_b.T, Whh_b.T, (bih_b + bhh_b)[None, :],
    )

    return _attn(
        hf, hb,
        W_attn[:H, :], W_attn[H:, :], v_attn[None, :],
        W_out[:, :H].T, W_out[:, H:].T, b_out[None, :],
    )


# speedup vs baseline: 3.9093x; 1.0865x over previous
import functools

import jax
import jax.numpy as jnp
from jax import lax
from jax.experimental import pallas as pl
from jax.experimental.pallas import tpu as pltpu
from jax.experimental.pallas import tpu_sc as plsc

VOCAB = 100000
EMBED = 256
HID = 512
H = HID // 2
G4 = 4 * H
ATTN = 256
TAG = 32
B = 128
L = 200

SC_CORES = 2
SC_SUBCORES = 16
NW = SC_CORES * SC_SUBCORES
ROWS = B * L
ROWS_PER_W = ROWS // NW
GCH = 80
NGCH = ROWS_PER_W // GCH

C = 8
NC = L // C

C2 = 40
NC2 = L // C2


def _sc_gather(emb, idx):
    mesh = plsc.VectorSubcoreMesh(core_axis_name="c", subcore_axis_name="s")

    @functools.partial(
        pl.kernel,
        mesh=mesh,
        out_type=jax.ShapeDtypeStruct((ROWS, EMBED), jnp.float32),
        scratch_types=[
            pltpu.VMEM((GCH,), jnp.int32),
            pltpu.VMEM((GCH, EMBED), jnp.float32),
            pltpu.SemaphoreType.DMA,
        ],
    )
    def gather_k(emb_hbm, idx_hbm, out_hbm, idx_v, rows_v, sem):
        wid = lax.axis_index("s") * SC_CORES + lax.axis_index("c")
        base = wid * ROWS_PER_W

        def body(c, carry):
            off = base + c * GCH
            pltpu.sync_copy(idx_hbm.at[pl.ds(off, GCH)], idx_v)
            pltpu.async_copy(emb_hbm.at[idx_v], rows_v, sem).wait()
            pltpu.sync_copy(rows_v, out_hbm.at[pl.ds(off, GCH)])
            return carry

        lax.fori_loop(0, NGCH, body, 0)

    return gather_k(emb, idx)


def _lstm_body(xef_ref, xeb_ref, wif_ref, whf_ref, bf_ref, wib_ref, whb_ref,
               bb_ref, hf_out, hb_out, hfs, cfs, hbs, cbs):
    g = pl.program_id(0)

    @pl.when(g == 0)
    def _():
        z = jnp.zeros((B, H), jnp.float32)
        hfs[...] = z
        cfs[...] = z
        hbs[...] = z
        cbs[...] = z

    whf = whf_ref[...]
    bf = bf_ref[...]
    whb = whb_ref[...]
    bb = bb_ref[...]

    hf = hfs[...]
    cf = cfs[...]
    hb = hbs[...]
    cb = cbs[...]

    wif = wif_ref[...]
    wib = wib_ref[...]

    def cell(x, h, c, wi, wh, b):
        gates = (jnp.dot(x, wi, preferred_element_type=jnp.float32) + b
                 + jnp.dot(h, wh, preferred_element_type=jnp.float32))
        i = jax.nn.sigmoid(gates[:, 0 * H:1 * H])
        f = jax.nn.sigmoid(gates[:, 1 * H:2 * H])
        gg = jnp.tanh(gates[:, 2 * H:3 * H])
        o = jax.nn.sigmoid(gates[:, 3 * H:4 * H])
        c_new = f * c + i * gg
        h_new = o * jnp.tanh(c_new)
        return h_new, c_new

    for t in range(C):
        tb = C - 1 - t
        hf, cf = cell(xef_ref[:, t, :], hf, cf, wif, whf, bf)
        hb, cb = cell(xeb_ref[:, tb, :], hb, cb, wib, whb, bb)
        hf_out[:, t, :] = hf
        hb_out[:, tb, :] = hb

    hfs[...] = hf
    cfs[...] = cf
    hbs[...] = hb
    cbs[...] = cb


def _lstm(xe, wif, whf, bf, wib, whb, bb):
    return pl.pallas_call(
        _lstm_body,
        grid=(NC,),
        in_specs=[
            pl.BlockSpec((B, C, EMBED), lambda g: (0, g, 0)),
            pl.BlockSpec((B, C, EMBED), lambda g: (0, NC - 1 - g, 0)),
            pl.BlockSpec((EMBED, G4), lambda g: (0, 0)),
            pl.BlockSpec((H, G4), lambda g: (0, 0)),
            pl.BlockSpec((1, G4), lambda g: (0, 0)),
            pl.BlockSpec((EMBED, G4), lambda g: (0, 0)),
            pl.BlockSpec((H, G4), lambda g: (0, 0)),
            pl.BlockSpec((1, G4), lambda g: (0, 0)),
        ],
        out_specs=[
            pl.BlockSpec((B, C, H), lambda g: (0, g, 0)),
            pl.BlockSpec((B, C, H), lambda g: (0, NC - 1 - g, 0)),
        ],
        out_shape=[
            jax.ShapeDtypeStruct((B, L, H), jnp.float32),
            jax.ShapeDtypeStruct((B, L, H), jnp.float32),
        ],
        scratch_shapes=[pltpu.VMEM((B, H), jnp.float32)] * 4,
        compiler_params=pltpu.CompilerParams(
            dimension_semantics=("arbitrary",)),
    )(xe, xe, wif, whf, bf, wib, whb, bb)


def _attn_body(hf_ref, hb_ref, waf_ref, wab_ref, v_ref, wof_ref, wob_ref,
               bo_ref, out_ref, accf_ref, accb_ref, z_ref):
    g = pl.program_id(0)

    @pl.when(g == 0)
    def _():
        accf_ref[...] = jnp.zeros((B, H), jnp.float32)
        accb_ref[...] = jnp.zeros((B, H), jnp.float32)
        z_ref[...] = jnp.zeros((B, 1), jnp.float32)

    waf = waf_ref[...]
    wab = wab_ref[...]
    v = v_ref[...]
    accf = accf_ref[...]
    accb = accb_ref[...]
    z = z_ref[...]
    for t in range(C2):
        hft = hf_ref[:, t, :]
        hbt = hb_ref[:, t, :]
        u = jnp.tanh(
            jnp.dot(hft, waf, preferred_element_type=jnp.float32)
            + jnp.dot(hbt, wab, preferred_element_type=jnp.float32))
        w = jnp.exp(jnp.sum(u * v, axis=1, keepdims=True))
        z = z + w
        accf = accf + w * hft
        accb = accb + w * hbt
    accf_ref[...] = accf
    accb_ref[...] = accb
    z_ref[...] = z

    @pl.when(g == NC2 - 1)
    def _():
        zr = 1.0 / z
        out_ref[...] = (
            jnp.dot(accf * zr, wof_ref[...],
                    preferred_element_type=jnp.float32)
            + jnp.dot(accb * zr, wob_ref[...],
                      preferred_element_type=jnp.float32)
            + bo_ref[...])


def _attn(hf, hb, waf, wab, v, wof, wob, bo):
    return pl.pallas_call(
        _attn_body,
        grid=(NC2,),
        in_specs=[
            pl.BlockSpec((B, C2, H), lambda g: (0, g, 0)),
            pl.BlockSpec((B, C2, H), lambda g: (0, g, 0)),
            pl.BlockSpec((H, ATTN), lambda g: (0, 0)),
            pl.BlockSpec((H, ATTN), lambda g: (0, 0)),
            pl.BlockSpec((1, ATTN), lambda g: (0, 0)),
            pl.BlockSpec((H, TAG), lambda g: (0, 0)),
            pl.BlockSpec((H, TAG), lambda g: (0, 0)),
            pl.BlockSpec((1, TAG), lambda g: (0, 0)),
        ],
        out_specs=pl.BlockSpec((B, TAG), lambda g: (0, 0)),
        out_shape=jax.ShapeDtypeStruct((B, TAG), jnp.float32),
        scratch_shapes=[
            pltpu.VMEM((B, H), jnp.float32),
            pltpu.VMEM((B, H), jnp.float32),
            pltpu.VMEM((B, 1), jnp.float32),
        ],
        compiler_params=pltpu.CompilerParams(
            dimension_semantics=("arbitrary",)),
    )(hf, hb, waf, wab, v, wof, wob, bo)


def kernel(x, emb, Wih_f, Whh_f, bih_f, bhh_f, Wih_b, Whh_b, bih_b, bhh_b,
           W_attn, v_attn, W_out, b_out):
    idx = x.reshape(-1).astype(jnp.int32)
    xe = _sc_gather(emb, idx).reshape(B, L, EMBED)

    hf, hb = _lstm(
        xe,
        Wih_f.T, Whh_f.T, (bih_f + bhh_f)[None, :],
        Wih_b.T, Whh_b.T, (bih_b + bhh_b)[None, :],
    )

    return _attn(
        hf, hb,
        W_attn[:H, :], W_attn[H:, :], v_attn[None, :],
        W_out[:, :H].T, W_out[:, H:].T, b_out[None, :],
    )


# speedup vs baseline: 5.0108x; 1.2817x over previous
import functools

import jax
import jax.numpy as jnp
from jax import lax
from jax.experimental import pallas as pl
from jax.experimental.pallas import tpu as pltpu
from jax.experimental.pallas import tpu_sc as plsc

VOCAB = 100000
EMBED = 256
HID = 512
H = HID // 2
G4 = 4 * H
ATTN = 256
TAG = 32
B = 128
L = 200

SC_CORES = 2
SC_SUBCORES = 16
NW = SC_CORES * SC_SUBCORES
ROWS = B * L
ROWS_PER_W = ROWS // NW
GCH = 80
NGCH = ROWS_PER_W // GCH

C = 8
NC = L // C

C2 = 40
NC2 = L // C2


def _sc_gather(emb, idx):
    mesh = plsc.VectorSubcoreMesh(core_axis_name="c", subcore_axis_name="s")

    @functools.partial(
        pl.kernel,
        mesh=mesh,
        out_type=jax.ShapeDtypeStruct((ROWS, EMBED), jnp.float32),
        scratch_types=[
            pltpu.VMEM((GCH,), jnp.int32),
            pltpu.VMEM((GCH, EMBED), jnp.float32),
            pltpu.SemaphoreType.DMA,
        ],
    )
    def gather_k(emb_hbm, idx_hbm, out_hbm, idx_v, rows_v, sem):
        wid = lax.axis_index("s") * SC_CORES + lax.axis_index("c")
        base = wid * ROWS_PER_W

        def body(c, carry):
            off = base + c * GCH
            pltpu.sync_copy(idx_hbm.at[pl.ds(off, GCH)], idx_v)
            pltpu.async_copy(emb_hbm.at[idx_v], rows_v, sem).wait()
            pltpu.sync_copy(rows_v, out_hbm.at[pl.ds(off, GCH)])
            return carry

        lax.fori_loop(0, NGCH, body, 0)

    return gather_k(emb, idx)


def _lstm_body(xef_ref, xeb_ref, wif_ref, whf_ref, bf_ref, wib_ref, whb_ref,
               bb_ref, hf_out, hb_out, hfs, cfs, hbs, cbs):
    g = pl.program_id(0)

    @pl.when(g == 0)
    def _():
        z = jnp.zeros((B, H), jnp.float32)
        hfs[...] = z
        cfs[...] = z
        hbs[...] = z
        cbs[...] = z

    whf = whf_ref[...]
    bf = bf_ref[...]
    whb = whb_ref[...]
    bb = bb_ref[...]

    hf = hfs[...]
    cf = cfs[...]
    hb = hbs[...]
    cb = cbs[...]

    wif = wif_ref[...]
    wib = wib_ref[...]

    def cell(x, h, c, wi, wh, b):
        gates = (jnp.dot(x, wi, preferred_element_type=jnp.float32) + b
                 + jnp.dot(h, wh, preferred_element_type=jnp.float32))
        i = jax.nn.sigmoid(gates[:, 0 * H:1 * H])
        f = jax.nn.sigmoid(gates[:, 1 * H:2 * H])
        gg = jnp.tanh(gates[:, 2 * H:3 * H])
        o = jax.nn.sigmoid(gates[:, 3 * H:4 * H])
        c_new = f * c + i * gg
        h_new = o * jnp.tanh(c_new)
        return h_new, c_new

    for t in range(C):
        tb = C - 1 - t
        hf, cf = cell(xef_ref[t], hf, cf, wif, whf, bf)
        hb, cb = cell(xeb_ref[tb], hb, cb, wib, whb, bb)
        hf_out[t] = hf
        hb_out[tb] = hb

    hfs[...] = hf
    cfs[...] = cf
    hbs[...] = hb
    cbs[...] = cb


def _lstm(xe, wif, whf, bf, wib, whb, bb):
    return pl.pallas_call(
        _lstm_body,
        grid=(NC,),
        in_specs=[
            pl.BlockSpec((C, B, EMBED), lambda g: (g, 0, 0)),
            pl.BlockSpec((C, B, EMBED), lambda g: (NC - 1 - g, 0, 0)),
            pl.BlockSpec((EMBED, G4), lambda g: (0, 0)),
            pl.BlockSpec((H, G4), lambda g: (0, 0)),
            pl.BlockSpec((1, G4), lambda g: (0, 0)),
            pl.BlockSpec((EMBED, G4), lambda g: (0, 0)),
            pl.BlockSpec((H, G4), lambda g: (0, 0)),
            pl.BlockSpec((1, G4), lambda g: (0, 0)),
        ],
        out_specs=[
            pl.BlockSpec((C, B, H), lambda g: (g, 0, 0)),
            pl.BlockSpec((C, B, H), lambda g: (NC - 1 - g, 0, 0)),
        ],
        out_shape=[
            jax.ShapeDtypeStruct((L, B, H), jnp.float32),
            jax.ShapeDtypeStruct((L, B, H), jnp.float32),
        ],
        scratch_shapes=[pltpu.VMEM((B, H), jnp.float32)] * 4,
        compiler_params=pltpu.CompilerParams(
            dimension_semantics=("arbitrary",)),
    )(xe, xe, wif, whf, bf, wib, whb, bb)


def _attn_body(hf_ref, hb_ref, waf_ref, wab_ref, v_ref, wof_ref, wob_ref,
               bo_ref, out_ref, accf_ref, accb_ref, z_ref):
    g = pl.program_id(0)

    @pl.when(g == 0)
    def _():
        accf_ref[...] = jnp.zeros((B, H), jnp.float32)
        accb_ref[...] = jnp.zeros((B, H), jnp.float32)
        z_ref[...] = jnp.zeros((B, 1), jnp.float32)

    waf = waf_ref[...]
    wab = wab_ref[...]
    v = v_ref[...]
    accf = accf_ref[...]
    accb = accb_ref[...]
    z = z_ref[...]
    for t in range(C2):
        hft = hf_ref[t]
        hbt = hb_ref[t]
        u = jnp.tanh(
            jnp.dot(hft, waf, preferred_element_type=jnp.float32)
            + jnp.dot(hbt, wab, preferred_element_type=jnp.float32))
        w = jnp.exp(jnp.sum(u * v, axis=1, keepdims=True))
        z = z + w
        accf = accf + w * hft
        accb = accb + w * hbt
    accf_ref[...] = accf
    accb_ref[...] = accb
    z_ref[...] = z

    @pl.when(g == NC2 - 1)
    def _():
        zr = 1.0 / z
        out_ref[...] = (
            jnp.dot(accf * zr, wof_ref[...],
                    preferred_element_type=jnp.float32)
            + jnp.dot(accb * zr, wob_ref[...],
                      preferred_element_type=jnp.float32)
            + bo_ref[...])


def _attn(hf, hb, waf, wab, v, wof, wob, bo):
    return pl.pallas_call(
        _attn_body,
        grid=(NC2,),
        in_specs=[
            pl.BlockSpec((C2, B, H), lambda g: (g, 0, 0)),
            pl.BlockSpec((C2, B, H), lambda g: (g, 0, 0)),
            pl.BlockSpec((H, ATTN), lambda g: (0, 0)),
            pl.BlockSpec((H, ATTN), lambda g: (0, 0)),
            pl.BlockSpec((1, ATTN), lambda g: (0, 0)),
            pl.BlockSpec((H, TAG), lambda g: (0, 0)),
            pl.BlockSpec((H, TAG), lambda g: (0, 0)),
            pl.BlockSpec((1, TAG), lambda g: (0, 0)),
        ],
        out_specs=pl.BlockSpec((B, TAG), lambda g: (0, 0)),
        out_shape=jax.ShapeDtypeStruct((B, TAG), jnp.float32),
        scratch_shapes=[
            pltpu.VMEM((B, H), jnp.float32),
            pltpu.VMEM((B, H), jnp.float32),
            pltpu.VMEM((B, 1), jnp.float32),
        ],
        compiler_params=pltpu.CompilerParams(
            dimension_semantics=("arbitrary",)),
    )(hf, hb, waf, wab, v, wof, wob, bo)


def kernel(x, emb, Wih_f, Whh_f, bih_f, bhh_f, Wih_b, Whh_b, bih_b, bhh_b,
           W_attn, v_attn, W_out, b_out):
    idx = x.T.reshape(-1).astype(jnp.int32)
    xe = _sc_gather(emb, idx).reshape(L, B, EMBED)

    hf, hb = _lstm(
        xe,
        Wih_f.T, Whh_f.T, (bih_f + bhh_f)[None, :],
        Wih_b.T, Whh_b.T, (bih_b + bhh_b)[None, :],
    )

    return _attn(
        hf, hb,
        W_attn[:H, :], W_attn[H:, :], v_attn[None, :],
        W_out[:, :H].T, W_out[:, H:].T, b_out[None, :],
    )


# speedup vs baseline: 5.3816x; 1.0740x over previous
import functools

import jax
import jax.numpy as jnp
from jax import lax
from jax.experimental import pallas as pl
from jax.experimental.pallas import tpu as pltpu
from jax.experimental.pallas import tpu_sc as plsc

VOCAB = 100000
EMBED = 256
HID = 512
H = HID // 2
G4 = 4 * H
ATTN = 256
TAG = 32
B = 128
L = 200

SC_CORES = 2
SC_SUBCORES = 16
NW = SC_CORES * SC_SUBCORES
ROWS = B * L
ROWS_PER_W = ROWS // NW
GCH = 80
NGCH = ROWS_PER_W // GCH

C = 8
NC = L // C

C2 = 40
NC2 = L // C2


def _sc_gather(emb, idx):
    mesh = plsc.VectorSubcoreMesh(core_axis_name="c", subcore_axis_name="s")

    @functools.partial(
        pl.kernel,
        mesh=mesh,
        out_type=jax.ShapeDtypeStruct((ROWS, EMBED), jnp.float32),
        scratch_types=[
            pltpu.VMEM((ROWS_PER_W,), jnp.int32),
            pltpu.VMEM((GCH, EMBED), jnp.float32),
            pltpu.VMEM((GCH, EMBED), jnp.float32),
            pltpu.SemaphoreType.DMA,
            pltpu.SemaphoreType.DMA,
            pltpu.SemaphoreType.DMA,
            pltpu.SemaphoreType.DMA,
        ],
    )
    def gather_k(emb_hbm, idx_hbm, out_hbm, idx_v, rows0, rows1,
                 gsem0, gsem1, wsem0, wsem1):
        wid = lax.axis_index("s") * SC_CORES + lax.axis_index("c")
        base = wid * ROWS_PER_W
        pltpu.sync_copy(idx_hbm.at[pl.ds(base, ROWS_PER_W)], idx_v)

        rows = (rows0, rows1)
        gsem = (gsem0, gsem1)
        wsem = (wsem0, wsem1)
        gathers = [None, None]
        writes = [None, None]
        for c in range(NGCH):
            s = c % 2
            if writes[s] is not None:
                writes[s].wait()
            gathers[s] = pltpu.async_copy(
                emb_hbm.at[idx_v.at[pl.ds(c * GCH, GCH)]], rows[s], gsem[s])
            if c > 0:
                sp = (c - 1) % 2
                gathers[sp].wait()
                writes[sp] = pltpu.async_copy(
                    rows[sp], out_hbm.at[pl.ds(base + (c - 1) * GCH, GCH)],
                    wsem[sp])
        s_last = (NGCH - 1) % 2
        gathers[s_last].wait()
        pltpu.sync_copy(rows[s_last],
                        out_hbm.at[pl.ds(base + (NGCH - 1) * GCH, GCH)])
        if writes[1 - s_last] is not None:
            writes[1 - s_last].wait()

    return gather_k(emb, idx)


def _lstm_body(xef_ref, xeb_ref, wif_ref, whf_ref, bf_ref, wib_ref, whb_ref,
               bb_ref, hf_out, hb_out, hfs, cfs, hbs, cbs):
    g = pl.program_id(0)

    @pl.when(g == 0)
    def _():
        z = jnp.zeros((B, H), jnp.float32)
        hfs[...] = z
        cfs[...] = z
        hbs[...] = z
        cbs[...] = z

    whf = whf_ref[...]
    bf = bf_ref[...]
    whb = whb_ref[...]
    bb = bb_ref[...]

    hf = hfs[...]
    cf = cfs[...]
    hb = hbs[...]
    cb = cbs[...]

    wif = wif_ref[...]
    wib = wib_ref[...]

    def cell(x, h, c, wi, wh, b):
        gates = (jnp.dot(x, wi, preferred_element_type=jnp.float32) + b
                 + jnp.dot(h, wh, preferred_element_type=jnp.float32))
        i = jax.nn.sigmoid(gates[:, 0 * H:1 * H])
        f = jax.nn.sigmoid(gates[:, 1 * H:2 * H])
        gg = jnp.tanh(gates[:, 2 * H:3 * H])
        o = jax.nn.sigmoid(gates[:, 3 * H:4 * H])
        c_new = f * c + i * gg
        h_new = o * jnp.tanh(c_new)
        return h_new, c_new

    for t in range(C):
        tb = C - 1 - t
        hf, cf = cell(xef_ref[t], hf, cf, wif, whf, bf)
        hb, cb = cell(xeb_ref[tb], hb, cb, wib, whb, bb)
        hf_out[t] = hf
        hb_out[tb] = hb

    hfs[...] = hf
    cfs[...] = cf
    hbs[...] = hb
    cbs[...] = cb


def _lstm(xe, wif, whf, bf, wib, whb, bb):
    return pl.pallas_call(
        _lstm_body,
        grid=(NC,),
        in_specs=[
            pl.BlockSpec((C, B, EMBED), lambda g: (g, 0, 0)),
            pl.BlockSpec((C, B, EMBED), lambda g: (NC - 1 - g, 0, 0)),
            pl.BlockSpec((EMBED, G4), lambda g: (0, 0)),
            pl.BlockSpec((H, G4), lambda g: (0, 0)),
            pl.BlockSpec((1, G4), lambda g: (0, 0)),
            pl.BlockSpec((EMBED, G4), lambda g: (0, 0)),
            pl.BlockSpec((H, G4), lambda g: (0, 0)),
            pl.BlockSpec((1, G4), lambda g: (0, 0)),
        ],
        out_specs=[
            pl.BlockSpec((C, B, H), lambda g: (g, 0, 0)),
            pl.BlockSpec((C, B, H), lambda g: (NC - 1 - g, 0, 0)),
        ],
        out_shape=[
            jax.ShapeDtypeStruct((L, B, H), jnp.float32),
            jax.ShapeDtypeStruct((L, B, H), jnp.float32),
        ],
        scratch_shapes=[pltpu.VMEM((B, H), jnp.float32)] * 4,
        compiler_params=pltpu.CompilerParams(
            dimension_semantics=("arbitrary",)),
    )(xe, xe, wif, whf, bf, wib, whb, bb)


def _attn_body(hf_ref, hb_ref, waf_ref, wab_ref, v_ref, wof_ref, wob_ref,
               bo_ref, out_ref, accf_ref, accb_ref, z_ref):
    g = pl.program_id(0)

    @pl.when(g == 0)
    def _():
        accf_ref[...] = jnp.zeros((B, H), jnp.float32)
        accb_ref[...] = jnp.zeros((B, H), jnp.float32)
        z_ref[...] = jnp.zeros((B, 1), jnp.float32)

    waf = waf_ref[...]
    wab = wab_ref[...]
    v = v_ref[...]
    accf = accf_ref[...]
    accb = accb_ref[...]
    z = z_ref[...]
    for t in range(C2):
        hft = hf_ref[t]
        hbt = hb_ref[t]
        u = jnp.tanh(
            jnp.dot(hft, waf, preferred_element_type=jnp.float32)
            + jnp.dot(hbt, wab, preferred_element_type=jnp.float32))
        w = jnp.exp(jnp.sum(u * v, axis=1, keepdims=True))
        z = z + w
        accf = accf + w * hft
        accb = accb + w * hbt
    accf_ref[...] = accf
    accb_ref[...] = accb
    z_ref[...] = z

    @pl.when(g == NC2 - 1)
    def _():
        zr = 1.0 / z
        out_ref[...] = (
            jnp.dot(accf * zr, wof_ref[...],
                    preferred_element_type=jnp.float32)
            + jnp.dot(accb * zr, wob_ref[...],
                      preferred_element_type=jnp.float32)
            + bo_ref[...])


def _attn(hf, hb, waf, wab, v, wof, wob, bo):
    return pl.pallas_call(
        _attn_body,
        grid=(NC2,),
        in_specs=[
            pl.BlockSpec((C2, B, H), lambda g: (g, 0, 0)),
            pl.BlockSpec((C2, B, H), lambda g: (g, 0, 0)),
            pl.BlockSpec((H, ATTN), lambda g: (0, 0)),
            pl.BlockSpec((H, ATTN), lambda g: (0, 0)),
            pl.BlockSpec((1, ATTN), lambda g: (0, 0)),
            pl.BlockSpec((H, TAG), lambda g: (0, 0)),
            pl.BlockSpec((H, TAG), lambda g: (0, 0)),
            pl.BlockSpec((1, TAG), lambda g: (0, 0)),
        ],
        out_specs=pl.BlockSpec((B, TAG), lambda g: (0, 0)),
        out_shape=jax.ShapeDtypeStruct((B, TAG), jnp.float32),
        scratch_shapes=[
            pltpu.VMEM((B, H), jnp.float32),
            pltpu.VMEM((B, H), jnp.float32),
            pltpu.VMEM((B, 1), jnp.float32),
        ],
        compiler_params=pltpu.CompilerParams(
            dimension_semantics=("arbitrary",)),
    )(hf, hb, waf, wab, v, wof, wob, bo)


def kernel(x, emb, Wih_f, Whh_f, bih_f, bhh_f, Wih_b, Whh_b, bih_b, bhh_b,
           W_attn, v_attn, W_out, b_out):
    idx = x.T.reshape(-1).astype(jnp.int32)
    xe = _sc_gather(emb, idx).reshape(L, B, EMBED)

    hf, hb = _lstm(
        xe,
        Wih_f.T, Whh_f.T, (bih_f + bhh_f)[None, :],
        Wih_b.T, Whh_b.T, (bih_b + bhh_b)[None, :],
    )

    return _attn(
        hf, hb,
        W_attn[:H, :], W_attn[H:, :], v_attn[None, :],
        W_out[:, :H].T, W_out[:, H:].T, b_out[None, :],
    )


# speedup vs baseline: 5.3915x; 1.0018x over previous
import functools

import jax
import jax.numpy as jnp
from jax import lax
from jax.experimental import pallas as pl
from jax.experimental.pallas import tpu as pltpu
from jax.experimental.pallas import tpu_sc as plsc

VOCAB = 100000
EMBED = 256
HID = 512
H = HID // 2
G4 = 4 * H
ATTN = 256
TAG = 32
B = 128
L = 200

SC_CORES = 2
SC_SUBCORES = 16
NW = SC_CORES * SC_SUBCORES
ROWS = B * L
ROWS_PER_W = ROWS // NW
GCH = 80
NGCH = ROWS_PER_W // GCH

C = 8
NC = L // C

C2 = 40
NC2 = L // C2


def _sc_gather(emb, idx):
    mesh = plsc.VectorSubcoreMesh(core_axis_name="c", subcore_axis_name="s")

    @functools.partial(
        pl.kernel,
        mesh=mesh,
        out_type=jax.ShapeDtypeStruct((ROWS, EMBED), jnp.float32),
        scratch_types=[
            pltpu.VMEM((ROWS_PER_W,), jnp.int32),
            pltpu.VMEM((GCH, EMBED), jnp.float32),
            pltpu.VMEM((GCH, EMBED), jnp.float32),
            pltpu.SemaphoreType.DMA,
            pltpu.SemaphoreType.DMA,
            pltpu.SemaphoreType.DMA,
            pltpu.SemaphoreType.DMA,
        ],
    )
    def gather_k(emb_hbm, idx_hbm, out_hbm, idx_v, rows0, rows1,
                 gsem0, gsem1, wsem0, wsem1):
        wid = lax.axis_index("s") * SC_CORES + lax.axis_index("c")
        base = wid * ROWS_PER_W
        pltpu.sync_copy(idx_hbm.at[pl.ds(base, ROWS_PER_W)], idx_v)

        rows = (rows0, rows1)
        gsem = (gsem0, gsem1)
        wsem = (wsem0, wsem1)
        gathers = [None, None]
        writes = [None, None]
        for c in range(NGCH):
            s = c % 2
            if writes[s] is not None:
                writes[s].wait()
            gathers[s] = pltpu.async_copy(
                emb_hbm.at[idx_v.at[pl.ds(c * GCH, GCH)]], rows[s], gsem[s])
            if c > 0:
                sp = (c - 1) % 2
                gathers[sp].wait()
                writes[sp] = pltpu.async_copy(
                    rows[sp], out_hbm.at[pl.ds(base + (c - 1) * GCH, GCH)],
                    wsem[sp])
        s_last = (NGCH - 1) % 2
        gathers[s_last].wait()
        pltpu.sync_copy(rows[s_last],
                        out_hbm.at[pl.ds(base + (NGCH - 1) * GCH, GCH)])
        if writes[1 - s_last] is not None:
            writes[1 - s_last].wait()

    return gather_k(emb, idx)


def _fused_body(xef_ref, xeb_ref, wif_ref, whf_ref, bf_ref, wib_ref,
                whb_ref, bb_ref, waf_ref, wab_ref, v_ref, wof_ref, wob_ref,
                bo_ref, out_ref, hfsc, hbsc, hfs, cfs, hbs, cbs,
                accf_ref, accb_ref, z_ref):
    p = pl.program_id(0)
    g = pl.program_id(1)

    @pl.when((p == 0) & (g == 0))
    def _():
        z = jnp.zeros((B, H), jnp.float32)
        hfs[...] = z
        cfs[...] = z
        hbs[...] = z
        cbs[...] = z
        accf_ref[...] = jnp.zeros((B, H), jnp.float32)
        accb_ref[...] = jnp.zeros((B, H), jnp.float32)
        z_ref[...] = jnp.zeros((B, 1), jnp.float32)

    @pl.when(p == 0)
    def _():
        whf = whf_ref[...]
        bf = bf_ref[...]
        whb = whb_ref[...]
        bb = bb_ref[...]
        wif = wif_ref[...]
        wib = wib_ref[...]

        hf = hfs[...]
        cf = cfs[...]
        hb = hbs[...]
        cb = cbs[...]

        def cell(x, h, c, wi, wh, b):
            gates = (jnp.dot(x, wi, preferred_element_type=jnp.float32) + b
                     + jnp.dot(h, wh, preferred_element_type=jnp.float32))
            i = jax.nn.sigmoid(gates[:, 0 * H:1 * H])
            f = jax.nn.sigmoid(gates[:, 1 * H:2 * H])
            gg = jnp.tanh(gates[:, 2 * H:3 * H])
            o = jax.nn.sigmoid(gates[:, 3 * H:4 * H])
            c_new = f * c + i * gg
            h_new = o * jnp.tanh(c_new)
            return h_new, c_new

        for t in range(C):
            tb = C - 1 - t
            hf, cf = cell(xef_ref[t], hf, cf, wif, whf, bf)
            hb, cb = cell(xeb_ref[tb], hb, cb, wib, whb, bb)
            hfsc[g, t] = hf.astype(jnp.bfloat16)
            hbsc[NC - 1 - g, tb] = hb.astype(jnp.bfloat16)

        hfs[...] = hf
        cfs[...] = cf
        hbs[...] = hb
        cbs[...] = cb

    @pl.when(p == 1)
    def _():
        waf = waf_ref[...]
        wab = wab_ref[...]
        v = v_ref[...]
        accf = accf_ref[...]
        accb = accb_ref[...]
        z = z_ref[...]
        for t in range(C):
            hft = hfsc[g, t].astype(jnp.float32)
            hbt = hbsc[g, t].astype(jnp.float32)
            u = jnp.tanh(
                jnp.dot(hft, waf, preferred_element_type=jnp.float32)
                + jnp.dot(hbt, wab, preferred_element_type=jnp.float32))
            w = jnp.exp(jnp.sum(u * v, axis=1, keepdims=True))
            z = z + w
            accf = accf + w * hft
            accb = accb + w * hbt
        accf_ref[...] = accf
        accb_ref[...] = accb
        z_ref[...] = z

        @pl.when(g == NC - 1)
        def _():
            zr = 1.0 / z
            out_ref[...] = (
                jnp.dot(accf * zr, wof_ref[...],
                        preferred_element_type=jnp.float32)
                + jnp.dot(accb * zr, wob_ref[...],
                          preferred_element_type=jnp.float32)
                + bo_ref[...])


def _fused(xe, wif, whf, bf, wib, whb, bb, waf, wab, v, wof, wob, bo):
    const2 = lambda p, g: (0, 0)
    return pl.pallas_call(
        _fused_body,
        grid=(2, NC),
        in_specs=[
            pl.BlockSpec((C, B, EMBED),
                         lambda p, g: (jnp.where(p == 0, g, NC - 1), 0, 0)),
            pl.BlockSpec((C, B, EMBED),
                         lambda p, g: (jnp.where(p == 0, NC - 1 - g, 0),
                                       0, 0)),
            pl.BlockSpec((EMBED, G4), const2),
            pl.BlockSpec((H, G4), const2),
            pl.BlockSpec((1, G4), const2),
            pl.BlockSpec((EMBED, G4), const2),
            pl.BlockSpec((H, G4), const2),
            pl.BlockSpec((1, G4), const2),
            pl.BlockSpec((H, ATTN), const2),
            pl.BlockSpec((H, ATTN), const2),
            pl.BlockSpec((1, ATTN), const2),
            pl.BlockSpec((H, TAG), const2),
            pl.BlockSpec((H, TAG), const2),
            pl.BlockSpec((1, TAG), const2),
        ],
        out_specs=pl.BlockSpec((B, TAG), const2),
        out_shape=jax.ShapeDtypeStruct((B, TAG), jnp.float32),
        scratch_shapes=[
            pltpu.VMEM((NC, C, B, H), jnp.bfloat16),
            pltpu.VMEM((NC, C, B, H), jnp.bfloat16),
            pltpu.VMEM((B, H), jnp.float32),
            pltpu.VMEM((B, H), jnp.float32),
            pltpu.VMEM((B, H), jnp.float32),
            pltpu.VMEM((B, H), jnp.float32),
            pltpu.VMEM((B, H), jnp.float32),
            pltpu.VMEM((B, H), jnp.float32),
            pltpu.VMEM((B, 1), jnp.float32),
        ],
        compiler_params=pltpu.CompilerParams(
            dimension_semantics=("arbitrary", "arbitrary")),
    )(xe, xe, wif, whf, bf, wib, whb, bb, waf, wab, v, wof, wob, bo)


def kernel(x, emb, Wih_f, Whh_f, bih_f, bhh_f, Wih_b, Whh_b, bih_b, bhh_b,
           W_attn, v_attn, W_out, b_out):
    idx = x.T.reshape(-1).astype(jnp.int32)
    xe = _sc_gather(emb, idx).reshape(L, B, EMBED)

    return _fused(
        xe,
        Wih_f.T, Whh_f.T, (bih_f + bhh_f)[None, :],
        Wih_b.T, Whh_b.T, (bih_b + bhh_b)[None, :],
        W_attn[:H, :], W_attn[H:, :], v_attn[None, :],
        W_out[:, :H].T, W_out[:, H:].T, b_out[None, :],
    )


# speedup vs baseline: 5.4505x; 1.0110x over previous
import functools

import jax
import jax.numpy as jnp
from jax import lax
from jax.experimental import pallas as pl
from jax.experimental.pallas import tpu as pltpu
from jax.experimental.pallas import tpu_sc as plsc

VOCAB = 100000
EMBED = 256
HID = 512
H = HID // 2
G4 = 4 * H
ATTN = 256
TAG = 32
B = 128
L = 200

SC_CORES = 2
SC_SUBCORES = 16
NW = SC_CORES * SC_SUBCORES
ROWS = B * L
ROWS_PER_W = ROWS // NW
GCH = 80
NGCH = ROWS_PER_W // GCH
NBUF = 4

C = 8
NC = L // C

C2 = 40
NC2 = L // C2


def _sc_gather(emb, idx):
    mesh = plsc.VectorSubcoreMesh(core_axis_name="c", subcore_axis_name="s")

    @functools.partial(
        pl.kernel,
        mesh=mesh,
        out_type=jax.ShapeDtypeStruct((ROWS, EMBED), jnp.float32),
        scratch_types=(
            [pltpu.VMEM((ROWS_PER_W,), jnp.int32)]
            + [pltpu.VMEM((GCH, EMBED), jnp.float32)] * NBUF
            + [pltpu.SemaphoreType.DMA] * (2 * NBUF)
        ),
    )
    def gather_k(emb_hbm, idx_hbm, out_hbm, idx_v, *bufs):
        rows = bufs[:NBUF]
        gsem = bufs[NBUF:2 * NBUF]
        wsem = bufs[2 * NBUF:]
        wid = lax.axis_index("s") * SC_CORES + lax.axis_index("c")
        base = wid * ROWS_PER_W
        pltpu.sync_copy(idx_hbm.at[pl.ds(base, ROWS_PER_W)], idx_v)

        gathers = [None] * NBUF
        writes = [None] * NBUF
        for c in range(NGCH):
            s = c % NBUF
            if writes[s] is not None:
                writes[s].wait()
            gathers[s] = pltpu.async_copy(
                emb_hbm.at[idx_v.at[pl.ds(c * GCH, GCH)]], rows[s], gsem[s])
            if c >= NBUF - 1:
                cp = c - (NBUF - 1)
                sp = cp % NBUF
                gathers[sp].wait()
                writes[sp] = pltpu.async_copy(
                    rows[sp], out_hbm.at[pl.ds(base + cp * GCH, GCH)],
                    wsem[sp])
        for cp in range(max(0, NGCH - (NBUF - 1)), NGCH):
            sp = cp % NBUF
            gathers[sp].wait()
            writes[sp] = pltpu.async_copy(
                rows[sp], out_hbm.at[pl.ds(base + cp * GCH, GCH)], wsem[sp])
        for w in writes:
            if w is not None:
                w.wait()

    return gather_k(emb, idx)


def _fused_body(xef_ref, xeb_ref, wif_ref, whf_ref, bf_ref, wib_ref,
                whb_ref, bb_ref, waf_ref, wab_ref, v_ref, wof_ref, wob_ref,
                bo_ref, out_ref, hfsc, hbsc, hfs, cfs, hbs, cbs,
                accf_ref, accb_ref, z_ref):
    p = pl.program_id(0)
    g = pl.program_id(1)

    @pl.when((p == 0) & (g == 0))
    def _():
        z = jnp.zeros((B, H), jnp.float32)
        hfs[...] = z
        cfs[...] = z
        hbs[...] = z
        cbs[...] = z
        accf_ref[...] = jnp.zeros((B, H), jnp.float32)
        accb_ref[...] = jnp.zeros((B, H), jnp.float32)
        z_ref[...] = jnp.zeros((B, 1), jnp.float32)

    @pl.when(p == 0)
    def _():
        whf = whf_ref[...]
        bf = bf_ref[...]
        whb = whb_ref[...]
        bb = bb_ref[...]
        wif = wif_ref[...]
        wib = wib_ref[...]

        hf = hfs[...]
        cf = cfs[...]
        hb = hbs[...]
        cb = cbs[...]

        def cell(x, h, c, wi, wh, b):
            gates = (jnp.dot(x, wi, preferred_element_type=jnp.float32) + b
                     + jnp.dot(h, wh, preferred_element_type=jnp.float32))
            i = jax.nn.sigmoid(gates[:, 0 * H:1 * H])
            f = jax.nn.sigmoid(gates[:, 1 * H:2 * H])
            gg = jnp.tanh(gates[:, 2 * H:3 * H])
            o = jax.nn.sigmoid(gates[:, 3 * H:4 * H])
            c_new = f * c + i * gg
            h_new = o * jnp.tanh(c_new)
            return h_new, c_new

        for t in range(C):
            tb = C - 1 - t
            hf, cf = cell(xef_ref[t], hf, cf, wif, whf, bf)
            hb, cb = cell(xeb_ref[tb], hb, cb, wib, whb, bb)
            hfsc[g, t] = hf.astype(jnp.bfloat16)
            hbsc[NC - 1 - g, tb] = hb.astype(jnp.bfloat16)

        hfs[...] = hf
        cfs[...] = cf
        hbs[...] = hb
        cbs[...] = cb

    @pl.when(p == 1)
    def _():
        waf = waf_ref[...]
        wab = wab_ref[...]
        v = v_ref[...]
        accf = accf_ref[...]
        accb = accb_ref[...]
        z = z_ref[...]
        for t in range(C):
            hft = hfsc[g, t].astype(jnp.float32)
            hbt = hbsc[g, t].astype(jnp.float32)
            u = jnp.tanh(
                jnp.dot(hft, waf, preferred_element_type=jnp.float32)
                + jnp.dot(hbt, wab, preferred_element_type=jnp.float32))
            w = jnp.exp(jnp.sum(u * v, axis=1, keepdims=True))
            z = z + w
            accf = accf + w * hft
            accb = accb + w * hbt
        accf_ref[...] = accf
        accb_ref[...] = accb
        z_ref[...] = z

        @pl.when(g == NC - 1)
        def _():
            zr = 1.0 / z
            out_ref[...] = (
                jnp.dot(accf * zr, wof_ref[...],
                        preferred_element_type=jnp.float32)
                + jnp.dot(accb * zr, wob_ref[...],
                          preferred_element_type=jnp.float32)
                + bo_ref[...])


def _fused(xe, wif, whf, bf, wib, whb, bb, waf, wab, v, wof, wob, bo):
    const2 = lambda p, g: (0, 0)
    return pl.pallas_call(
        _fused_body,
        grid=(2, NC),
        in_specs=[
            pl.BlockSpec((C, B, EMBED),
                         lambda p, g: (jnp.where(p == 0, g, NC - 1), 0, 0)),
            pl.BlockSpec((C, B, EMBED),
                         lambda p, g: (jnp.where(p == 0, NC - 1 - g, 0),
                                       0, 0)),
            pl.BlockSpec((EMBED, G4), const2),
            pl.BlockSpec((H, G4), const2),
            pl.BlockSpec((1, G4), const2),
            pl.BlockSpec((EMBED, G4), const2),
            pl.BlockSpec((H, G4), const2),
            pl.BlockSpec((1, G4), const2),
            pl.BlockSpec((H, ATTN), const2),
            pl.BlockSpec((H, ATTN), const2),
            pl.BlockSpec((1, ATTN), const2),
            pl.BlockSpec((H, TAG), const2),
            pl.BlockSpec((H, TAG), const2),
            pl.BlockSpec((1, TAG), const2),
        ],
        out_specs=pl.BlockSpec((B, TAG), const2),
        out_shape=jax.ShapeDtypeStruct((B, TAG), jnp.float32),
        scratch_shapes=[
            pltpu.VMEM((NC, C, B, H), jnp.bfloat16),
            pltpu.VMEM((NC, C, B, H), jnp.bfloat16),
            pltpu.VMEM((B, H), jnp.float32),
            pltpu.VMEM((B, H), jnp.float32),
            pltpu.VMEM((B, H), jnp.float32),
            pltpu.VMEM((B, H), jnp.float32),
            pltpu.VMEM((B, H), jnp.float32),
            pltpu.VMEM((B, H), jnp.float32),
            pltpu.VMEM((B, 1), jnp.float32),
        ],
        compiler_params=pltpu.CompilerParams(
            dimension_semantics=("arbitrary", "arbitrary")),
    )(xe, xe, wif, whf, bf, wib, whb, bb, waf, wab, v, wof, wob, bo)


def kernel(x, emb, Wih_f, Whh_f, bih_f, bhh_f, Wih_b, Whh_b, bih_b, bhh_b,
           W_attn, v_attn, W_out, b_out):
    idx = x.T.reshape(-1).astype(jnp.int32)
    xe = _sc_gather(emb, idx).reshape(L, B, EMBED)

    return _fused(
        xe,
        Wih_f.T, Whh_f.T, (bih_f + bhh_f)[None, :],
        Wih_b.T, Whh_b.T, (bih_b + bhh_b)[None, :],
        W_attn[:H, :], W_attn[H:, :], v_attn[None, :],
        W_out[:, :H].T, W_out[:, H:].T, b_out[None, :],
    )


# speedup vs baseline: 6.0776x; 1.1150x over previous
import functools

import jax
import jax.numpy as jnp
from jax import lax
from jax.experimental import pallas as pl
from jax.experimental.pallas import tpu as pltpu
from jax.experimental.pallas import tpu_sc as plsc

VOCAB = 100000
EMBED = 256
HID = 512
H = HID // 2
G4 = 4 * H
ATTN = 256
TAG = 32
B = 128
L = 200

SC_CORES = 2
SC_SUBCORES = 16
NW = SC_CORES * SC_SUBCORES
ROWS = B * L
ROWS_PER_W = ROWS // NW
GCH = 80
NGCH = ROWS_PER_W // GCH
NBUF = 4

C = 40
NC = L // C

C2 = 40
NC2 = L // C2


def _sc_gather(emb, idx):
    mesh = plsc.VectorSubcoreMesh(core_axis_name="c", subcore_axis_name="s")

    @functools.partial(
        pl.kernel,
        mesh=mesh,
        out_type=jax.ShapeDtypeStruct((ROWS, EMBED), jnp.float32),
        scratch_types=(
            [pltpu.VMEM((ROWS_PER_W,), jnp.int32)]
            + [pltpu.VMEM((GCH, EMBED), jnp.float32)] * NBUF
            + [pltpu.SemaphoreType.DMA] * (2 * NBUF)
        ),
    )
    def gather_k(emb_hbm, idx_hbm, out_hbm, idx_v, *bufs):
        rows = bufs[:NBUF]
        gsem = bufs[NBUF:2 * NBUF]
        wsem = bufs[2 * NBUF:]
        wid = lax.axis_index("s") * SC_CORES + lax.axis_index("c")
        base = wid * ROWS_PER_W
        pltpu.sync_copy(idx_hbm.at[pl.ds(base, ROWS_PER_W)], idx_v)

        gathers = [None] * NBUF
        writes = [None] * NBUF
        for c in range(NGCH):
            s = c % NBUF
            if writes[s] is not None:
                writes[s].wait()
            gathers[s] = pltpu.async_copy(
                emb_hbm.at[idx_v.at[pl.ds(c * GCH, GCH)]], rows[s], gsem[s])
            if c >= NBUF - 1:
                cp = c - (NBUF - 1)
                sp = cp % NBUF
                gathers[sp].wait()
                writes[sp] = pltpu.async_copy(
                    rows[sp], out_hbm.at[pl.ds(base + cp * GCH, GCH)],
                    wsem[sp])
        for cp in range(max(0, NGCH - (NBUF - 1)), NGCH):
            sp = cp % NBUF
            gathers[sp].wait()
            writes[sp] = pltpu.async_copy(
                rows[sp], out_hbm.at[pl.ds(base + cp * GCH, GCH)], wsem[sp])
        for w in writes:
            if w is not None:
                w.wait()

    return gather_k(emb, idx)


def _fused_body(xef_ref, xeb_ref, wif_ref, whf_ref, bf_ref, wib_ref,
                whb_ref, bb_ref, waf_ref, wab_ref, v_ref, wof_ref, wob_ref,
                bo_ref, out_ref, hfsc, hbsc, hfs, cfs, hbs, cbs,
                accf_ref, accb_ref, z_ref):
    p = pl.program_id(0)
    g = pl.program_id(1)

    @pl.when((p == 0) & (g == 0))
    def _():
        z = jnp.zeros((B, H), jnp.float32)
        hfs[...] = z
        cfs[...] = z
        hbs[...] = z
        cbs[...] = z
        accf_ref[...] = jnp.zeros((B, H), jnp.float32)
        accb_ref[...] = jnp.zeros((B, H), jnp.float32)
        z_ref[...] = jnp.zeros((B, 1), jnp.float32)

    @pl.when(p == 0)
    def _():
        whf = whf_ref[...]
        bf = bf_ref[...]
        whb = whb_ref[...]
        bb = bb_ref[...]
        wif = wif_ref[...]
        wib = wib_ref[...]

        hf = hfs[...]
        cf = cfs[...]
        hb = hbs[...]
        cb = cbs[...]

        def cell(x, h, c, wi, wh, b):
            gates = (jnp.dot(x, wi, preferred_element_type=jnp.float32) + b
                     + jnp.dot(h, wh, preferred_element_type=jnp.float32))
            i = jax.nn.sigmoid(gates[:, 0 * H:1 * H])
            f = jax.nn.sigmoid(gates[:, 1 * H:2 * H])
            gg = jnp.tanh(gates[:, 2 * H:3 * H])
            o = jax.nn.sigmoid(gates[:, 3 * H:4 * H])
            c_new = f * c + i * gg
            h_new = o * jnp.tanh(c_new)
            return h_new, c_new

        for t in range(C):
            tb = C - 1 - t
            hf, cf = cell(xef_ref[t], hf, cf, wif, whf, bf)
            hb, cb = cell(xeb_ref[tb], hb, cb, wib, whb, bb)
            hfsc[g, t] = hf.astype(jnp.bfloat16)
            hbsc[NC - 1 - g, tb] = hb.astype(jnp.bfloat16)

        hfs[...] = hf
        cfs[...] = cf
        hbs[...] = hb
        cbs[...] = cb

    @pl.when(p == 1)
    def _():
        waf = waf_ref[...]
        wab = wab_ref[...]
        v = v_ref[...]
        accf = accf_ref[...]
        accb = accb_ref[...]
        z = z_ref[...]
        for t in range(C):
            hft = hfsc[g, t].astype(jnp.float32)
            hbt = hbsc[g, t].astype(jnp.float32)
            u = jnp.tanh(
                jnp.dot(hft, waf, preferred_element_type=jnp.float32)
                + jnp.dot(hbt, wab, preferred_element_type=jnp.float32))
            w = jnp.exp(jnp.sum(u * v, axis=1, keepdims=True))
            z = z + w
            accf = accf + w * hft
            accb = accb + w * hbt
        accf_ref[...] = accf
        accb_ref[...] = accb
        z_ref[...] = z

        @pl.when(g == NC - 1)
        def _():
            zr = 1.0 / z
            out_ref[...] = (
                jnp.dot(accf * zr, wof_ref[...],
                        preferred_element_type=jnp.float32)
                + jnp.dot(accb * zr, wob_ref[...],
                          preferred_element_type=jnp.float32)
                + bo_ref[...])


def _fused(xe, wif, whf, bf, wib, whb, bb, waf, wab, v, wof, wob, bo):
    const2 = lambda p, g: (0, 0)
    return pl.pallas_call(
        _fused_body,
        grid=(2, NC),
        in_specs=[
            pl.BlockSpec((C, B, EMBED),
                         lambda p, g: (jnp.where(p == 0, g, NC - 1), 0, 0)),
            pl.BlockSpec((C, B, EMBED),
                         lambda p, g: (jnp.where(p == 0, NC - 1 - g, 0),
                                       0, 0)),
            pl.BlockSpec((EMBED, G4), const2),
            pl.BlockSpec((H, G4), const2),
            pl.BlockSpec((1, G4), const2),
            pl.BlockSpec((EMBED, G4), const2),
            pl.BlockSpec((H, G4), const2),
            pl.BlockSpec((1, G4), const2),
            pl.BlockSpec((H, ATTN), const2),
            pl.BlockSpec((H, ATTN), const2),
            pl.BlockSpec((1, ATTN), const2),
            pl.BlockSpec((H, TAG), const2),
            pl.BlockSpec((H, TAG), const2),
            pl.BlockSpec((1, TAG), const2),
        ],
        out_specs=pl.BlockSpec((B, TAG), const2),
        out_shape=jax.ShapeDtypeStruct((B, TAG), jnp.float32),
        scratch_shapes=[
            pltpu.VMEM((NC, C, B, H), jnp.bfloat16),
            pltpu.VMEM((NC, C, B, H), jnp.bfloat16),
            pltpu.VMEM((B, H), jnp.float32),
            pltpu.VMEM((B, H), jnp.float32),
            pltpu.VMEM((B, H), jnp.float32),
            pltpu.VMEM((B, H), jnp.float32),
            pltpu.VMEM((B, H), jnp.float32),
            pltpu.VMEM((B, H), jnp.float32),
            pltpu.VMEM((B, 1), jnp.float32),
        ],
        compiler_params=pltpu.CompilerParams(
            dimension_semantics=("arbitrary", "arbitrary")),
    )(xe, xe, wif, whf, bf, wib, whb, bb, waf, wab, v, wof, wob, bo)


def kernel(x, emb, Wih_f, Whh_f, bih_f, bhh_f, Wih_b, Whh_b, bih_b, bhh_b,
           W_attn, v_attn, W_out, b_out):
    idx = x.T.reshape(-1).astype(jnp.int32)
    xe = _sc_gather(emb, idx).reshape(L, B, EMBED)

    return _fused(
        xe,
        Wih_f.T, Whh_f.T, (bih_f + bhh_f)[None, :],
        Wih_b.T, Whh_b.T, (bih_b + bhh_b)[None, :],
        W_attn[:H, :], W_attn[H:, :], v_attn[None, :],
        W_out[:, :H].T, W_out[:, H:].T, b_out[None, :],
    )
